# baseline (speedup 1.0000x reference)
import jax
import jax.numpy as jnp
from jax.experimental import pallas as pl
from jax.experimental.pallas import tpu as pltpu

_N = 10000
_F = 128
_BM1 = 200
_BM2 = 400


def _layer1_kernel(x_ref, w1_ref, b1_ref, w2_ref, g_ref,
                   s2_ref, g16_ref, s1_scr):
    @pl.when(pl.program_id(0) == 0)
    def _():
        s1_scr[...] = jnp.dot(
            x_ref[...], w1_ref[...], preferred_element_type=jnp.float32
        )

    gb = g_ref[...]
    g16_ref[...] = gb.astype(jnp.bfloat16)
    acc = jnp.dot(gb, s1_scr[...], preferred_element_type=jnp.float32)
    h = jnp.maximum(acc + b1_ref[...], 0.0)
    s2_ref[...] = jnp.dot(
        h, w2_ref[...], preferred_element_type=jnp.float32
    ).astype(jnp.bfloat16)


def _layer2_kernel(s2_ref, b2_ref, g16_ref, out_ref):
    acc = jnp.dot(g16_ref[...], s2_ref[...], preferred_element_type=jnp.float32)
    out_ref[...] = acc + b2_ref[...]


def kernel(g, x, W1, b1, W2, b2):
    s2, g16 = pl.pallas_call(
        _layer1_kernel,
        grid=(_N // _BM1,),
        in_specs=[
            pl.BlockSpec((_N, _F), lambda i: (0, 0)),
            pl.BlockSpec((_F, _F), lambda i: (0, 0)),
            pl.BlockSpec((1, _F), lambda i: (0, 0)),
            pl.BlockSpec((_F, _F), lambda i: (0, 0)),
            pl.BlockSpec((_BM1, _N), lambda i: (i, 0)),
        ],
        out_specs=[
            pl.BlockSpec((_BM1, _F), lambda i: (i, 0)),
            pl.BlockSpec((_BM1, _N), lambda i: (i, 0)),
        ],
        out_shape=[
            jax.ShapeDtypeStruct((_N, _F), jnp.bfloat16),
            jax.ShapeDtypeStruct((_N, _N), jnp.bfloat16),
        ],
        scratch_shapes=[pltpu.VMEM((_N, _F), jnp.float32)],
        compiler_params=pltpu.CompilerParams(
            dimension_semantics=("arbitrary",),
        ),
    )(x, W1, b1.reshape(1, _F), W2, g)

    out = pl.pallas_call(
        _layer2_kernel,
        grid=(_N // _BM2,),
        in_specs=[
            pl.BlockSpec((_N, _F), lambda i: (0, 0)),
            pl.BlockSpec((1, _F), lambda i: (0, 0)),
            pl.BlockSpec((_BM2, _N), lambda i: (i, 0)),
        ],
        out_specs=pl.BlockSpec((_BM2, _F), lambda i: (i, 0)),
        out_shape=jax.ShapeDtypeStruct((_N, _F), jnp.float32),
        compiler_params=pltpu.CompilerParams(
            dimension_semantics=("parallel",),
        ),
    )(s2, b2.reshape(1, _F), g16)

    return out


# speedup vs baseline: 1.0962x; 1.0962x over previous
import jax
import jax.numpy as jnp
from jax.experimental import pallas as pl
from jax.experimental.pallas import tpu as pltpu

_N = 10000
_F = 128
_BM = 400
_NB = _N // _BM


def _gcn_kernel(x_ref, w1_ref, b1_ref, w2_ref, b2_ref, g_ref,
                out_ref, s1_scr, s2_scr):
    i = pl.program_id(0)

    @pl.when(i == 0)
    def _():
        s1_scr[...] = jnp.dot(
            x_ref[...], w1_ref[...], preferred_element_type=jnp.float32
        )

    @pl.when(i < _NB)
    def _():
        acc = jnp.dot(
            g_ref[...], s1_scr[...], preferred_element_type=jnp.float32
        )
        h = jnp.maximum(acc + b1_ref[...], 0.0)
        s2_scr[pl.ds(i * _BM, _BM), :] = jnp.dot(
            h, w2_ref[...], preferred_element_type=jnp.float32
        )

    @pl.when(i >= _NB)
    def _():
        out_ref[...] = jnp.dot(
            g_ref[...], s2_scr[...], preferred_element_type=jnp.float32
        ) + b2_ref[...]


def kernel(g, x, W1, b1, W2, b2):
    return pl.pallas_call(
        _gcn_kernel,
        grid=(2 * _NB,),
        in_specs=[
            pl.BlockSpec((_N, _F), lambda i: (0, 0)),
            pl.BlockSpec((_F, _F), lambda i: (0, 0)),
            pl.BlockSpec((1, _F), lambda i: (0, 0)),
            pl.BlockSpec((_F, _F), lambda i: (0, 0)),
            pl.BlockSpec((1, _F), lambda i: (0, 0)),
            pl.BlockSpec((_BM, _N), lambda i: (i % _NB, 0)),
        ],
        out_specs=pl.BlockSpec(
            (_BM, _F), lambda i: ((i // _NB) * (i - _NB), 0)
        ),
        out_shape=jax.ShapeDtypeStruct((_N, _F), jnp.float32),
        scratch_shapes=[
            pltpu.VMEM((_N, _F), jnp.float32),
            pltpu.VMEM((_N, _F), jnp.float32),
        ],
        compiler_params=pltpu.CompilerParams(
            dimension_semantics=("arbitrary",),
        ),
    )(x, W1, b1.reshape(1, _F), W2, b2.reshape(1, _F), g)


# speedup vs baseline: 1.0979x; 1.0016x over previous
import jax
import jax.numpy as jnp
from jax.experimental import pallas as pl
from jax.experimental.pallas import tpu as pltpu

_N = 10000
_F = 128
_BM = 400
_NB = _N // _BM


def _gcn_kernel(x_ref, w1_ref, b1_ref, w2_ref, b2_ref, g_ref,
                out_ref, s1_scr, s2_scr):
    i = pl.program_id(0)

    @pl.when(i == 0)
    def _():
        s1_scr[...] = jnp.dot(
            x_ref[...], w1_ref[...], preferred_element_type=jnp.float32
        )

    @pl.when(i < _NB)
    def _():
        acc = jnp.dot(
            g_ref[...], s1_scr[...], preferred_element_type=jnp.float32
        )
        h = jnp.maximum(acc + b1_ref[...], 0.0)
        s2_scr[pl.ds(i * _BM, _BM), :] = jnp.dot(
            h, w2_ref[...], preferred_element_type=jnp.float32
        )

    @pl.when(i >= _NB)
    def _():
        out_ref[...] = jnp.dot(
            g_ref[...], s2_scr[...], preferred_element_type=jnp.float32
        ) + b2_ref[...]


def kernel(g, x, W1, b1, W2, b2):
    return pl.pallas_call(
        _gcn_kernel,
        grid=(2 * _NB,),
        in_specs=[
            pl.BlockSpec((_N, _F), lambda i: (0, 0)),
            pl.BlockSpec((_F, _F), lambda i: (0, 0)),
            pl.BlockSpec((1, _F), lambda i: (0, 0)),
            pl.BlockSpec((_F, _F), lambda i: (0, 0)),
            pl.BlockSpec((1, _F), lambda i: (0, 0)),
            pl.BlockSpec(
                (_BM, _N),
                lambda i: (
                    (1 - i // _NB) * (i % _NB)
                    + (i // _NB) * (2 * _NB - 1 - i),
                    0,
                ),
            ),
        ],
        out_specs=pl.BlockSpec(
            (_BM, _F),
            lambda i: (
                (1 - i // _NB) * (_NB - 1)
                + (i // _NB) * (2 * _NB - 1 - i),
                0,
            ),
        ),
        out_shape=jax.ShapeDtypeStruct((_N, _F), jnp.float32),
        scratch_shapes=[
            pltpu.VMEM((_N, _F), jnp.float32),
            pltpu.VMEM((_N, _F), jnp.float32),
        ],
        compiler_params=pltpu.CompilerParams(
            dimension_semantics=("arbitrary",),
        ),
    )(x, W1, b1.reshape(1, _F), W2, b2.reshape(1, _F), g)
